# baseline (speedup 1.0000x reference)
import functools

import jax
import jax.numpy as jnp
from jax.experimental import pallas as pl
from jax.experimental.pallas import tpu as pltpu


def _stats_kernel(x_ref, m_ref, mem_ref, sx_ref, cs_ref, dn_ref, *, spb, d):
    s = pl.program_id(0) % spb
    xv = x_ref[...]
    mv = m_ref[...]
    scores = jax.lax.dot_general(
        xv, mem_ref[...], (((1,), (1,)), ((), ())),
        preferred_element_type=jnp.float32) * (1.0 / (d ** 0.5))
    mx = jnp.max(scores, axis=1, keepdims=True)
    e = jnp.exp(scores - mx)
    attn = e / jnp.sum(e, axis=1, keepdims=True)
    sx = jnp.sum(xv * mv, axis=0)[None, None, :]
    cs = jnp.sum(attn * mv, axis=0)[None, None, :]
    dn = jnp.sum(mv).reshape(1, 1, 1)

    @pl.when(s == 0)
    def _():
        sx_ref[...] = sx
        cs_ref[...] = cs
        dn_ref[...] = dn

    @pl.when(s > 0)
    def _():
        sx_ref[...] += sx
        cs_ref[...] += cs
        dn_ref[...] += dn


def _first_argmax_onehot(logits):
    mv = jnp.max(logits, axis=1, keepdims=True)
    iota = jax.lax.broadcasted_iota(jnp.int32, logits.shape, 1)
    cand = jnp.where(logits == mv, iota, logits.shape[1])
    am = jnp.min(cand, axis=1, keepdims=True)
    return (iota == am).astype(jnp.float32)


def _router_kernel(sx_ref, cs_ref, dn_ref, mem_ref, wr_ref, br_ref,
                   wop_ref, wtask_ref, opemb_ref, taskemb_ref,
                   ba1_ref, bl1_ref, ba2_ref, bl2_ref,
                   biasA_ref, biasL_ref, bias2_ref, w_ref):
    sx = sx_ref[:, 0, :]
    cs = cs_ref[:, 0, :]
    dn = jnp.maximum(dn_ref[:, 0, :], 1e-6)
    pooled_raw = sx / dn
    memsum = jnp.dot(cs, mem_ref[...], preferred_element_type=jnp.float32)
    pooled = pooled_raw + 0.2 * (memsum / dn)
    logits = jnp.dot(pooled, wr_ref[...],
                     preferred_element_type=jnp.float32) + br_ref[...]
    lmax = jnp.max(logits, axis=1, keepdims=True)
    le = jnp.exp(logits - lmax)
    w = le / jnp.sum(le, axis=1, keepdims=True)
    w_ref[...] = w
    opl = jnp.dot(pooled_raw, wop_ref[...], preferred_element_type=jnp.float32)
    tkl = jnp.dot(pooled_raw, wtask_ref[...], preferred_element_type=jnp.float32)
    ohA = _first_argmax_onehot(opl)
    ohL = _first_argmax_onehot(tkl)
    biasA = jnp.dot(ohA, opemb_ref[...],
                    preferred_element_type=jnp.float32) + ba1_ref[...]
    biasL = jnp.dot(ohL, taskemb_ref[...],
                    preferred_element_type=jnp.float32) + bl1_ref[...]
    biasA_ref[...] = biasA[:, None, :]
    biasL_ref[...] = biasL[:, None, :]
    bias2 = w[:, 0:1] * ba2_ref[...] + w[:, 1:2] * bl2_ref[...]
    bias2_ref[...] = bias2[:, None, :]


def _mlp_kernel(x_ref, wa1_ref, wl1_ref, wa2_ref, wl2_ref,
                bA_ref, bL_ref, b2_ref, w_ref,
                wf_ref, bf_ref, wc1_ref, wc2_ref, o_ref, acc_ref, *, tpb):
    t = pl.program_id(0)
    k = pl.program_id(1)
    nk = pl.num_programs(1)
    b = t // tpb
    xv = x_ref[...]
    ha = jnp.dot(xv, wa1_ref[...],
                 preferred_element_type=jnp.float32) + bA_ref[0, 0, :][None, :]
    ha = (jnp.maximum(ha, 0.0) * w_ref[b, 0]).astype(jnp.bfloat16)
    hl = jnp.dot(xv, wl1_ref[...],
                 preferred_element_type=jnp.float32) + bL_ref[0, 0, :][None, :]
    hl = (jax.nn.gelu(hl, approximate=True) * w_ref[b, 1]).astype(jnp.bfloat16)
    contrib = jnp.dot(ha, wa2_ref[...], preferred_element_type=jnp.float32)
    contrib = contrib + jnp.dot(hl, wl2_ref[...],
                                preferred_element_type=jnp.float32)

    @pl.when(k == 0)
    def _():
        acc_ref[...] = contrib

    @pl.when(k > 0)
    def _():
        acc_ref[...] += contrib

    @pl.when(k == nk - 1)
    def _():
        fused = acc_ref[...] + b2_ref[0, 0, :][None, :]
        y = jnp.dot(fused.astype(jnp.bfloat16), wf_ref[...],
                    preferred_element_type=jnp.float32) + bf_ref[...]
        th = jnp.tanh(jnp.dot(y.astype(jnp.bfloat16), wc1_ref[...],
                              preferred_element_type=jnp.float32))
        ctx = jnp.dot(th.astype(jnp.bfloat16), wc2_ref[...],
                      preferred_element_type=jnp.float32)
        o_ref[...] = (y + ctx) * 0.5


def kernel(hidden_states, attention_mask, memory, Wr, br, Wop, Wtask, OpEmb,
           TaskEmb, Wa1, ba1, Wa2, ba2, Wl1, bl1, Wl2, bl2, Wf, bf, Wc1, Wc2,
           interpret=False):
    B, S, D = hidden_states.shape
    M = memory.shape[0]
    DF = Wa1.shape[1]
    T = B * S
    TS = 512
    TT = 1024
    spb = S // TS
    tpb = S // TT

    x = hidden_states.reshape(T, D)
    mask2 = attention_mask.reshape(T, 1)

    sx, cs, dn = pl.pallas_call(
        functools.partial(_stats_kernel, spb=spb, d=D),
        grid=(T // TS,),
        in_specs=[
            pl.BlockSpec((TS, D), lambda t: (t, 0)),
            pl.BlockSpec((TS, 1), lambda t: (t, 0)),
            pl.BlockSpec((M, D), lambda t: (0, 0)),
        ],
        out_specs=[
            pl.BlockSpec((1, 1, D), lambda t, _spb=spb: (t // _spb, 0, 0)),
            pl.BlockSpec((1, 1, M), lambda t, _spb=spb: (t // _spb, 0, 0)),
            pl.BlockSpec((1, 1, 1), lambda t, _spb=spb: (t // _spb, 0, 0)),
        ],
        out_shape=[
            jax.ShapeDtypeStruct((B, 1, D), jnp.float32),
            jax.ShapeDtypeStruct((B, 1, M), jnp.float32),
            jax.ShapeDtypeStruct((B, 1, 1), jnp.float32),
        ],
        compiler_params=pltpu.CompilerParams(
            dimension_semantics=("arbitrary",)),
        interpret=interpret,
    )(x, mask2, memory)

    biasA, biasL, bias2, w = pl.pallas_call(
        _router_kernel,
        out_shape=[
            jax.ShapeDtypeStruct((B, 1, DF), jnp.float32),
            jax.ShapeDtypeStruct((B, 1, DF), jnp.float32),
            jax.ShapeDtypeStruct((B, 1, D), jnp.float32),
            jax.ShapeDtypeStruct((B, 2), jnp.float32),
        ],
        interpret=interpret,
    )(sx, cs, dn, memory, Wr, br.reshape(1, 2), Wop, Wtask, OpEmb, TaskEmb,
      ba1.reshape(1, DF), bl1.reshape(1, DF), ba2.reshape(1, D),
      bl2.reshape(1, D))

    KD = 1024
    xb = x.astype(jnp.bfloat16)
    out = pl.pallas_call(
        functools.partial(_mlp_kernel, tpb=tpb),
        grid=(T // TT, DF // KD),
        in_specs=[
            pl.BlockSpec((TT, D), lambda t, k: (t, 0)),
            pl.BlockSpec((D, KD), lambda t, k: (0, k)),
            pl.BlockSpec((D, KD), lambda t, k: (0, k)),
            pl.BlockSpec((KD, D), lambda t, k: (k, 0)),
            pl.BlockSpec((KD, D), lambda t, k: (k, 0)),
            pl.BlockSpec((1, 1, KD), lambda t, k, _tpb=tpb: (t // _tpb, 0, k)),
            pl.BlockSpec((1, 1, KD), lambda t, k, _tpb=tpb: (t // _tpb, 0, k)),
            pl.BlockSpec((1, 1, D), lambda t, k, _tpb=tpb: (t // _tpb, 0, 0)),
            pl.BlockSpec(memory_space=pltpu.SMEM),
            pl.BlockSpec((D, D), lambda t, k: (0, 0)),
            pl.BlockSpec((1, D), lambda t, k: (0, 0)),
            pl.BlockSpec((D, D), lambda t, k: (0, 0)),
            pl.BlockSpec((D, D), lambda t, k: (0, 0)),
        ],
        out_specs=pl.BlockSpec((TT, D), lambda t, k: (t, 0)),
        out_shape=jax.ShapeDtypeStruct((T, D), jnp.float32),
        scratch_shapes=[pltpu.VMEM((TT, D), jnp.float32)],
        compiler_params=pltpu.CompilerParams(
            dimension_semantics=("arbitrary", "arbitrary")),
        interpret=interpret,
    )(xb, Wa1.astype(jnp.bfloat16), Wl1.astype(jnp.bfloat16),
      Wa2.astype(jnp.bfloat16), Wl2.astype(jnp.bfloat16),
      biasA, biasL, bias2, w,
      Wf.astype(jnp.bfloat16), bf.reshape(1, D),
      Wc1.astype(jnp.bfloat16), Wc2.astype(jnp.bfloat16))

    return out.reshape(B, S, D)


# speedup vs baseline: 1.0770x; 1.0770x over previous
import functools

import jax
import jax.numpy as jnp
from jax.experimental import pallas as pl
from jax.experimental.pallas import tpu as pltpu


def _stats_kernel(x_ref, m_ref, mem_ref, sx_ref, cs_ref, dn_ref, *, spb, d):
    s = pl.program_id(0) % spb
    xv = x_ref[...]
    mv = m_ref[...]
    scores = jax.lax.dot_general(
        xv, mem_ref[...], (((1,), (1,)), ((), ())),
        preferred_element_type=jnp.float32) * (1.0 / (d ** 0.5))
    mx = jnp.max(scores, axis=1, keepdims=True)
    e = jnp.exp(scores - mx)
    attn = e / jnp.sum(e, axis=1, keepdims=True)
    sx = jnp.sum(xv * mv, axis=0)[None, None, :]
    cs = jnp.sum(attn * mv, axis=0)[None, None, :]
    dn = jnp.sum(mv).reshape(1, 1, 1)

    @pl.when(s == 0)
    def _():
        sx_ref[...] = sx
        cs_ref[...] = cs
        dn_ref[...] = dn

    @pl.when(s > 0)
    def _():
        sx_ref[...] += sx
        cs_ref[...] += cs
        dn_ref[...] += dn


def _first_argmax_onehot(logits):
    mv = jnp.max(logits, axis=1, keepdims=True)
    iota = jax.lax.broadcasted_iota(jnp.int32, logits.shape, 1)
    cand = jnp.where(logits == mv, iota, logits.shape[1])
    am = jnp.min(cand, axis=1, keepdims=True)
    return (iota == am).astype(jnp.float32)


def _router_kernel(sx_ref, cs_ref, dn_ref, mem_ref, wr_ref, br_ref,
                   wop_ref, wtask_ref, opemb_ref, taskemb_ref,
                   ba1_ref, bl1_ref, ba2_ref, bl2_ref,
                   biasA_ref, biasL_ref, bias2_ref, w_ref):
    sx = sx_ref[:, 0, :]
    cs = cs_ref[:, 0, :]
    dn = jnp.maximum(dn_ref[:, 0, :], 1e-6)
    pooled_raw = sx / dn
    memsum = jnp.dot(cs, mem_ref[...], preferred_element_type=jnp.float32)
    pooled = pooled_raw + 0.2 * (memsum / dn)
    logits = jnp.dot(pooled, wr_ref[...],
                     preferred_element_type=jnp.float32) + br_ref[...]
    lmax = jnp.max(logits, axis=1, keepdims=True)
    le = jnp.exp(logits - lmax)
    w = le / jnp.sum(le, axis=1, keepdims=True)
    w_ref[...] = w
    opl = jnp.dot(pooled_raw, wop_ref[...], preferred_element_type=jnp.float32)
    tkl = jnp.dot(pooled_raw, wtask_ref[...], preferred_element_type=jnp.float32)
    ohA = _first_argmax_onehot(opl)
    ohL = _first_argmax_onehot(tkl)
    biasA = jnp.dot(ohA, opemb_ref[...],
                    preferred_element_type=jnp.float32) + ba1_ref[...]
    biasL = jnp.dot(ohL, taskemb_ref[...],
                    preferred_element_type=jnp.float32) + bl1_ref[...]
    biasA_ref[...] = biasA[:, None, :].astype(jnp.bfloat16)
    biasL_ref[...] = biasL[:, None, :].astype(jnp.bfloat16)
    bias2 = w[:, 0:1] * ba2_ref[...] + w[:, 1:2] * bl2_ref[...]
    bias2_ref[...] = bias2[:, None, :]


def _mlp_kernel(x_ref, wa1_ref, wl1_ref, wa2_ref, wl2_ref,
                bA_ref, bL_ref, b2_ref, w_ref,
                wf_ref, bf_ref, wc1_ref, wc2_ref, o_ref, *, tpb, kd):
    t = pl.program_id(0)
    b = t // tpb
    xv = x_ref[...].astype(jnp.bfloat16)
    df = bA_ref.shape[-1]
    w0 = w_ref[b, 0].astype(jnp.bfloat16)
    w1 = w_ref[b, 1].astype(jnp.bfloat16)
    fused = b2_ref[0, 0, :][None, :] * jnp.ones_like(x_ref[..., :1])
    for kk in range(df // kd):
        sl = slice(kk * kd, (kk + 1) * kd)
        ha = jnp.dot(xv, wa1_ref[:, sl],
                     preferred_element_type=jnp.float32).astype(jnp.bfloat16)
        ha = jnp.maximum(ha + bA_ref[0, 0, sl][None, :],
                         jnp.bfloat16(0.0)) * w0
        hl = jnp.dot(xv, wl1_ref[:, sl],
                     preferred_element_type=jnp.float32).astype(jnp.bfloat16)
        hl = jax.nn.gelu(hl + bL_ref[0, 0, sl][None, :], approximate=True) * w1
        fused = fused + jnp.dot(ha, wa2_ref[sl, :],
                                preferred_element_type=jnp.float32)
        fused = fused + jnp.dot(hl, wl2_ref[sl, :],
                                preferred_element_type=jnp.float32)
    y = jnp.dot(fused.astype(jnp.bfloat16), wf_ref[...],
                preferred_element_type=jnp.float32) + bf_ref[...]
    th = jnp.tanh(jnp.dot(y.astype(jnp.bfloat16), wc1_ref[...],
                          preferred_element_type=jnp.float32))
    ctx = jnp.dot(th.astype(jnp.bfloat16), wc2_ref[...],
                  preferred_element_type=jnp.float32)
    o_ref[...] = (y + ctx) * 0.5


def kernel(hidden_states, attention_mask, memory, Wr, br, Wop, Wtask, OpEmb,
           TaskEmb, Wa1, ba1, Wa2, ba2, Wl1, bl1, Wl2, bl2, Wf, bf, Wc1, Wc2,
           interpret=False):
    B, S, D = hidden_states.shape
    M = memory.shape[0]
    DF = Wa1.shape[1]
    T = B * S
    TS = 512
    TT = 512
    spb = S // TS
    tpb = S // TT

    x = hidden_states.reshape(T, D)
    mask2 = attention_mask.reshape(T, 1)

    sx, cs, dn = pl.pallas_call(
        functools.partial(_stats_kernel, spb=spb, d=D),
        grid=(T // TS,),
        in_specs=[
            pl.BlockSpec((TS, D), lambda t: (t, 0)),
            pl.BlockSpec((TS, 1), lambda t: (t, 0)),
            pl.BlockSpec((M, D), lambda t: (0, 0)),
        ],
        out_specs=[
            pl.BlockSpec((1, 1, D), lambda t, _spb=spb: (t // _spb, 0, 0)),
            pl.BlockSpec((1, 1, M), lambda t, _spb=spb: (t // _spb, 0, 0)),
            pl.BlockSpec((1, 1, 1), lambda t, _spb=spb: (t // _spb, 0, 0)),
        ],
        out_shape=[
            jax.ShapeDtypeStruct((B, 1, D), jnp.float32),
            jax.ShapeDtypeStruct((B, 1, M), jnp.float32),
            jax.ShapeDtypeStruct((B, 1, 1), jnp.float32),
        ],
        compiler_params=pltpu.CompilerParams(
            dimension_semantics=("arbitrary",)),
        interpret=interpret,
    )(x, mask2, memory)

    biasA, biasL, bias2, w = pl.pallas_call(
        _router_kernel,
        out_shape=[
            jax.ShapeDtypeStruct((B, 1, DF), jnp.bfloat16),
            jax.ShapeDtypeStruct((B, 1, DF), jnp.bfloat16),
            jax.ShapeDtypeStruct((B, 1, D), jnp.float32),
            jax.ShapeDtypeStruct((B, 2), jnp.float32),
        ],
        interpret=interpret,
    )(sx, cs, dn, memory, Wr, br.reshape(1, 2), Wop, Wtask, OpEmb, TaskEmb,
      ba1.reshape(1, DF), bl1.reshape(1, DF), ba2.reshape(1, D),
      bl2.reshape(1, D))

    out = pl.pallas_call(
        functools.partial(_mlp_kernel, tpb=tpb, kd=1024),
        grid=(T // TT,),
        in_specs=[
            pl.BlockSpec((TT, D), lambda t: (t, 0)),
            pl.BlockSpec((D, DF), lambda t: (0, 0)),
            pl.BlockSpec((D, DF), lambda t: (0, 0)),
            pl.BlockSpec((DF, D), lambda t: (0, 0)),
            pl.BlockSpec((DF, D), lambda t: (0, 0)),
            pl.BlockSpec((1, 1, DF), lambda t, _tpb=tpb: (t // _tpb, 0, 0)),
            pl.BlockSpec((1, 1, DF), lambda t, _tpb=tpb: (t // _tpb, 0, 0)),
            pl.BlockSpec((1, 1, D), lambda t, _tpb=tpb: (t // _tpb, 0, 0)),
            pl.BlockSpec(memory_space=pltpu.SMEM),
            pl.BlockSpec((D, D), lambda t: (0, 0)),
            pl.BlockSpec((1, D), lambda t: (0, 0)),
            pl.BlockSpec((D, D), lambda t: (0, 0)),
            pl.BlockSpec((D, D), lambda t: (0, 0)),
        ],
        out_specs=pl.BlockSpec((TT, D), lambda t: (t, 0)),
        out_shape=jax.ShapeDtypeStruct((T, D), jnp.float32),
        compiler_params=pltpu.CompilerParams(
            dimension_semantics=("arbitrary",)),
        interpret=interpret,
    )(x, Wa1.astype(jnp.bfloat16), Wl1.astype(jnp.bfloat16),
      Wa2.astype(jnp.bfloat16), Wl2.astype(jnp.bfloat16),
      biasA, biasL, bias2, w,
      Wf.astype(jnp.bfloat16), bf.reshape(1, D),
      Wc1.astype(jnp.bfloat16), Wc2.astype(jnp.bfloat16))

    return out.reshape(B, S, D)


# speedup vs baseline: 1.0780x; 1.0009x over previous
import functools

import jax
import jax.numpy as jnp
from jax.experimental import pallas as pl
from jax.experimental.pallas import tpu as pltpu


def _stats_kernel(x_ref, m_ref, mem_ref, sx_ref, cs_ref, dn_ref, *, spb, d):
    s = pl.program_id(0) % spb
    xv = x_ref[...]
    mv = m_ref[...]
    scores = jax.lax.dot_general(
        xv, mem_ref[...], (((1,), (1,)), ((), ())),
        preferred_element_type=jnp.float32) * (1.0 / (d ** 0.5))
    mx = jnp.max(scores, axis=1, keepdims=True)
    e = jnp.exp(scores - mx)
    attn = e / jnp.sum(e, axis=1, keepdims=True)
    sx = jnp.sum(xv * mv, axis=0)[None, None, :]
    cs = jnp.sum(attn * mv, axis=0)[None, None, :]
    dn = jnp.sum(mv).reshape(1, 1, 1)

    @pl.when(s == 0)
    def _():
        sx_ref[...] = sx
        cs_ref[...] = cs
        dn_ref[...] = dn

    @pl.when(s > 0)
    def _():
        sx_ref[...] += sx
        cs_ref[...] += cs
        dn_ref[...] += dn


def _first_argmax_onehot(logits):
    mv = jnp.max(logits, axis=1, keepdims=True)
    iota = jax.lax.broadcasted_iota(jnp.int32, logits.shape, 1)
    cand = jnp.where(logits == mv, iota, logits.shape[1])
    am = jnp.min(cand, axis=1, keepdims=True)
    return (iota == am).astype(jnp.float32)


def _router_kernel(sx_ref, cs_ref, dn_ref, mem_ref, wr_ref, br_ref,
                   wop_ref, wtask_ref, opemb_ref, taskemb_ref,
                   ba1_ref, bl1_ref, ba2_ref, bl2_ref,
                   biasA_ref, biasL_ref, bias2_ref, w_ref):
    sx = sx_ref[:, 0, :]
    cs = cs_ref[:, 0, :]
    dn = jnp.maximum(dn_ref[:, 0, :], 1e-6)
    pooled_raw = sx / dn
    memsum = jnp.dot(cs, mem_ref[...], preferred_element_type=jnp.float32)
    pooled = pooled_raw + 0.2 * (memsum / dn)
    logits = jnp.dot(pooled, wr_ref[...],
                     preferred_element_type=jnp.float32) + br_ref[...]
    lmax = jnp.max(logits, axis=1, keepdims=True)
    le = jnp.exp(logits - lmax)
    w = le / jnp.sum(le, axis=1, keepdims=True)
    w_ref[...] = w
    opl = jnp.dot(pooled_raw, wop_ref[...], preferred_element_type=jnp.float32)
    tkl = jnp.dot(pooled_raw, wtask_ref[...], preferred_element_type=jnp.float32)
    ohA = _first_argmax_onehot(opl)
    ohL = _first_argmax_onehot(tkl)
    biasA = jnp.dot(ohA, opemb_ref[...],
                    preferred_element_type=jnp.float32) + ba1_ref[...]
    biasL = jnp.dot(ohL, taskemb_ref[...],
                    preferred_element_type=jnp.float32) + bl1_ref[...]
    biasA_ref[...] = biasA[:, None, :].astype(jnp.bfloat16)
    biasL_ref[...] = biasL[:, None, :].astype(jnp.bfloat16)
    bias2 = w[:, 0:1] * ba2_ref[...] + w[:, 1:2] * bl2_ref[...]
    bias2_ref[...] = bias2[:, None, :]


def _mlp_kernel(x_ref, wa1_ref, wl1_ref, wa2_ref, wl2_ref,
                bA_ref, bL_ref, b2_ref, w_ref,
                wf_ref, bf_ref, wc1_ref, wc2_ref, o_ref, *, tpb, kd):
    t = pl.program_id(0)
    b = t // tpb
    xv = x_ref[...].astype(jnp.bfloat16)
    df = bA_ref.shape[-1]
    w0 = w_ref[b, 0].astype(jnp.bfloat16)
    w1 = w_ref[b, 1].astype(jnp.bfloat16)
    fused = b2_ref[0, 0, :][None, :] * jnp.ones_like(x_ref[..., :1])
    for kk in range(df // kd):
        sl = slice(kk * kd, (kk + 1) * kd)
        ha = jnp.dot(xv, wa1_ref[:, sl],
                     preferred_element_type=jnp.float32).astype(jnp.bfloat16)
        ha = jnp.maximum(ha + bA_ref[0, 0, sl][None, :],
                         jnp.bfloat16(0.0)) * w0
        hl = jnp.dot(xv, wl1_ref[:, sl],
                     preferred_element_type=jnp.float32).astype(jnp.bfloat16)
        hl = jax.nn.gelu(hl + bL_ref[0, 0, sl][None, :], approximate=True) * w1
        fused = fused + jnp.dot(ha, wa2_ref[sl, :],
                                preferred_element_type=jnp.float32)
        fused = fused + jnp.dot(hl, wl2_ref[sl, :],
                                preferred_element_type=jnp.float32)
    y = jnp.dot(fused.astype(jnp.bfloat16), wf_ref[...],
                preferred_element_type=jnp.float32) + bf_ref[...]
    th = jnp.tanh(jnp.dot(y.astype(jnp.bfloat16), wc1_ref[...],
                          preferred_element_type=jnp.float32))
    ctx = jnp.dot(th.astype(jnp.bfloat16), wc2_ref[...],
                  preferred_element_type=jnp.float32)
    o_ref[...] = (y + ctx) * 0.5


def kernel(hidden_states, attention_mask, memory, Wr, br, Wop, Wtask, OpEmb,
           TaskEmb, Wa1, ba1, Wa2, ba2, Wl1, bl1, Wl2, bl2, Wf, bf, Wc1, Wc2,
           interpret=False):
    B, S, D = hidden_states.shape
    M = memory.shape[0]
    DF = Wa1.shape[1]
    T = B * S
    TS = 512
    TT = 512
    spb = S // TS
    tpb = S // TT

    x = hidden_states.reshape(T, D)
    mask2 = attention_mask.reshape(T, 1)

    sx, cs, dn = pl.pallas_call(
        functools.partial(_stats_kernel, spb=spb, d=D),
        grid=(T // TS,),
        in_specs=[
            pl.BlockSpec((TS, D), lambda t: (t, 0)),
            pl.BlockSpec((TS, 1), lambda t: (t, 0)),
            pl.BlockSpec((M, D), lambda t: (0, 0)),
        ],
        out_specs=[
            pl.BlockSpec((1, 1, D), lambda t, _spb=spb: (t // _spb, 0, 0)),
            pl.BlockSpec((1, 1, M), lambda t, _spb=spb: (t // _spb, 0, 0)),
            pl.BlockSpec((1, 1, 1), lambda t, _spb=spb: (t // _spb, 0, 0)),
        ],
        out_shape=[
            jax.ShapeDtypeStruct((B, 1, D), jnp.float32),
            jax.ShapeDtypeStruct((B, 1, M), jnp.float32),
            jax.ShapeDtypeStruct((B, 1, 1), jnp.float32),
        ],
        compiler_params=pltpu.CompilerParams(
            dimension_semantics=("arbitrary",)),
        interpret=interpret,
    )(x, mask2, memory)

    biasA, biasL, bias2, w = pl.pallas_call(
        _router_kernel,
        out_shape=[
            jax.ShapeDtypeStruct((B, 1, DF), jnp.bfloat16),
            jax.ShapeDtypeStruct((B, 1, DF), jnp.bfloat16),
            jax.ShapeDtypeStruct((B, 1, D), jnp.float32),
            jax.ShapeDtypeStruct((B, 2), jnp.float32),
        ],
        interpret=interpret,
    )(sx, cs, dn, memory, Wr, br.reshape(1, 2), Wop, Wtask, OpEmb, TaskEmb,
      ba1.reshape(1, DF), bl1.reshape(1, DF), ba2.reshape(1, D),
      bl2.reshape(1, D))

    out = pl.pallas_call(
        functools.partial(_mlp_kernel, tpb=tpb, kd=2048),
        grid=(T // TT,),
        in_specs=[
            pl.BlockSpec((TT, D), lambda t: (t, 0)),
            pl.BlockSpec((D, DF), lambda t: (0, 0)),
            pl.BlockSpec((D, DF), lambda t: (0, 0)),
            pl.BlockSpec((DF, D), lambda t: (0, 0)),
            pl.BlockSpec((DF, D), lambda t: (0, 0)),
            pl.BlockSpec((1, 1, DF), lambda t, _tpb=tpb: (t // _tpb, 0, 0)),
            pl.BlockSpec((1, 1, DF), lambda t, _tpb=tpb: (t // _tpb, 0, 0)),
            pl.BlockSpec((1, 1, D), lambda t, _tpb=tpb: (t // _tpb, 0, 0)),
            pl.BlockSpec(memory_space=pltpu.SMEM),
            pl.BlockSpec((D, D), lambda t: (0, 0)),
            pl.BlockSpec((1, D), lambda t: (0, 0)),
            pl.BlockSpec((D, D), lambda t: (0, 0)),
            pl.BlockSpec((D, D), lambda t: (0, 0)),
        ],
        out_specs=pl.BlockSpec((TT, D), lambda t: (t, 0)),
        out_shape=jax.ShapeDtypeStruct((T, D), jnp.float32),
        compiler_params=pltpu.CompilerParams(
            dimension_semantics=("arbitrary",)),
        interpret=interpret,
    )(x, Wa1.astype(jnp.bfloat16), Wl1.astype(jnp.bfloat16),
      Wa2.astype(jnp.bfloat16), Wl2.astype(jnp.bfloat16),
      biasA, biasL, bias2, w,
      Wf.astype(jnp.bfloat16), bf.reshape(1, D),
      Wc1.astype(jnp.bfloat16), Wc2.astype(jnp.bfloat16))

    return out.reshape(B, S, D)
